# baseline (speedup 1.0000x reference)
import jax
import jax.numpy as jnp
from jax.experimental import pallas as pl

N = 10000
E = 320000
CUR = 128
HID = 64
OUT = 32
H = 6


def _add_self_loops(edge_index, edge_weight=None, n=N):
    loop = jnp.arange(n, dtype=edge_index.dtype)
    ei = jnp.concatenate([edge_index, jnp.stack([loop, loop])], axis=1)
    if edge_weight is None:
        return ei
    ew = jnp.concatenate([edge_weight, jnp.ones((n,), edge_weight.dtype)], axis=0)
    return ei, ew


def _gcn_norm(edge_index, edge_weight, n=N, add_self_loops=True):
    if add_self_loops:
        edge_index, edge_weight = _add_self_loops(edge_index, edge_weight, n)
    row, col = edge_index[0], edge_index[1]
    deg = jax.ops.segment_sum(edge_weight, col, num_segments=n)
    dis = jnp.where(deg > 0, jax.lax.rsqrt(jnp.where(deg > 0, deg, 1.0)), 0.0)
    return edge_index, dis[row] * edge_weight * dis[col]


def _gat(h, edge_index, W, att_src, att_dst, bias):
    ei = _add_self_loops(edge_index)
    xp = (h @ W).reshape(N, H, OUT)
    a_src = jnp.sum(xp * att_src[None], axis=-1)
    a_dst = jnp.sum(xp * att_dst[None], axis=-1)
    src, dst = ei[0], ei[1]
    alpha = jax.nn.leaky_relu(a_src[src] + a_dst[dst], 0.2)
    amax = jax.ops.segment_max(alpha, dst, num_segments=N)
    amax = jax.lax.stop_gradient(jnp.where(jnp.isfinite(amax), amax, 0.0))
    ex = jnp.exp(alpha - amax[dst])
    denom = jax.ops.segment_sum(ex, dst, num_segments=N)
    att = ex / (denom[dst] + 1e-16)
    out = jax.ops.segment_sum(xp[src] * att[:, :, None], dst, num_segments=N)
    return out.reshape(N, H * OUT) + bias


def _gcn(h, edge_index, edge_weight, W, b):
    ei, norm = _gcn_norm(edge_index, edge_weight)
    xp = h @ W
    out = jax.ops.segment_sum(xp[ei[0]] * norm[:, None], ei[1], num_segments=N)
    return out + b


def _arma(h1, edge_index, edge_weight, init_W, root_W, b):
    ei, norm = _gcn_norm(edge_index, edge_weight, add_self_loops=False)
    out = h1 @ init_W
    out = jax.ops.segment_sum(out[ei[0]] * norm[:, None], ei[1], num_segments=N)
    return jax.nn.relu(out + h1 @ root_W + b)


def _fuse_body(h1_ref, h2_ref, h3_ref, h4_ref, o_ref):
    o_ref[...] = jnp.tanh(
        jnp.concatenate([h1_ref[...], h2_ref[...], h3_ref[...], h4_ref[...]], axis=1)
    )


def kernel(x, edge_index, edge_weight, W_pre, b_pre, gat_W, att_src, att_dst, gat_b,
           gcn0_W, gcn0_b, gcn1_W, gcn1_b, arma_init_W, arma_root_W, arma_b):
    h = x @ W_pre + b_pre
    h1 = jax.nn.leaky_relu(_gat(h, edge_index, gat_W, att_src, att_dst, gat_b), 0.01)
    h2 = jax.nn.leaky_relu(_gcn(h, edge_index, edge_weight, gcn0_W, gcn0_b), 0.01)
    h3 = jax.nn.leaky_relu(_gcn(h, edge_index, edge_weight, gcn1_W, gcn1_b), 0.01)
    h4 = jax.nn.leaky_relu(_arma(h1, edge_index, edge_weight, arma_init_W, arma_root_W, arma_b), 0.01)
    return pl.pallas_call(
        _fuse_body,
        out_shape=jax.ShapeDtypeStruct((N, H * OUT + 3 * OUT), jnp.float32),
    )(h1, h2, h3, h4)


# speedup vs baseline: 25.5626x; 25.5626x over previous
import functools

import jax
import jax.numpy as jnp
from jax import lax
from jax.experimental import pallas as pl
from jax.experimental.pallas import tpu as pltpu
from jax.experimental.pallas import tpu_sc as plsc

N = 10000
E = 320000
CUR = 128
HID = 64
OUT = 32
H = 6
HO = H * OUT

NP = 10112
ROWS_PT = NP // 16
NW = 32
EPW = E // NW
CH = 80
NCH = EPW // CH

_MESH = plsc.VectorSubcoreMesh(core_axis_name="c", subcore_axis_name="s")


def _take(vec, i):
    return vec.at[jnp.full((16,), i, jnp.int32)].get(mode="promise_in_bounds")


def _zero_acc(z, acc, s):
    pltpu.sync_copy(z.at[pl.ds(s * ROWS_PT, ROWS_PT)],
                    acc.at[pl.ds(s * ROWS_PT, ROWS_PT)])


def _copy_out(acc, out, c, s):
    pltpu.sync_copy(acc.at[pl.ds(s * ROWS_PT, ROWS_PT)],
                    out.at[c, pl.ds(s * ROWS_PT, ROWS_PT)])


def _sc_edge1_body(src, dst, atab, exw_out,
                   idx_s, idx_d, rs_v, rd_v, ex_v, sem):
    c = lax.axis_index("c")
    s = lax.axis_index("s")
    w = c * 16 + s

    lane = lax.iota(jnp.int32, 16)
    shift6 = jnp.where(lane < 10, lane + 6, 15)

    def chunk(g, carry):
        base = w * EPW + g * CH
        pltpu.sync_copy(src.at[pl.ds(base, CH)], idx_s)
        pltpu.sync_copy(dst.at[pl.ds(base, CH)], idx_d)
        pltpu.async_copy(atab.at[idx_s], rs_v, sem).wait()
        pltpu.async_copy(atab.at[idx_d], rd_v, sem).wait()

        def edge(e, cc):
            av = rs_v[e, pl.ds(0, 16)]
            bv = rd_v[e, pl.ds(0, 16)]
            bsh = bv.at[shift6].get(mode="promise_in_bounds")
            al = av + bsh
            al = jnp.where(al > 0, al, al * 0.2)
            ex_v[e, pl.ds(0, 16)] = jnp.where(lane < 6, jnp.exp(al), 0.0)
            return cc

        lax.fori_loop(0, CH, edge, 0)
        pltpu.sync_copy(ex_v, exw_out.at[pl.ds(base, CH)])
        return carry

    lax.fori_loop(0, NCH, chunk, 0)


_sc_edge1 = functools.partial(
    pl.kernel,
    out_type=jax.ShapeDtypeStruct((E, 16), jnp.float32),
    mesh=_MESH,
    scratch_types=[
        pltpu.VMEM((CH,), jnp.int32),
        pltpu.VMEM((CH,), jnp.int32),
        pltpu.VMEM((CH, 128), jnp.float32),
        pltpu.VMEM((CH, 128), jnp.float32),
        pltpu.VMEM((CH, 16), jnp.float32),
        pltpu.SemaphoreType.DMA,
    ],
)(_sc_edge1_body)


def _sc_gat_a_body(src, dst, exw, xph, zw, part_out,
                   idx_s, idx_d, ex_v, rows_v, msg_v, acc, sem):
    c = lax.axis_index("c")
    s = lax.axis_index("s")
    w = c * 16 + s
    _zero_acc(zw, acc, s)
    plsc.subcore_barrier()

    def chunk(g, carry):
        base = w * EPW + g * CH
        pltpu.sync_copy(src.at[pl.ds(base, CH)], idx_s)
        pltpu.sync_copy(dst.at[pl.ds(base, CH)], idx_d)
        pltpu.sync_copy(exw.at[pl.ds(base, CH)], ex_v)
        pltpu.async_copy(xph.at[idx_s], rows_v, sem).wait()

        def edge(e, cc):
            exv = ex_v[e, pl.ds(0, 16)]
            for h in range(4):
                f = _take(exv, h)
                o = h * 32
                msg_v[e, pl.ds(o, 16)] = rows_v[e, pl.ds(o, 16)] * f
                msg_v[e, pl.ds(o + 16, 16)] = rows_v[e, pl.ds(o + 16, 16)] * f
            return cc

        lax.fori_loop(0, CH, edge, 0)
        pltpu.sync_copy(msg_v, acc.at[idx_d], add=True)
        return carry

    lax.fori_loop(0, NCH, chunk, 0)
    plsc.subcore_barrier()
    _copy_out(acc, part_out, c, s)


_sc_gat_a = functools.partial(
    pl.kernel,
    out_type=jax.ShapeDtypeStruct((2, NP, 128), jnp.float32),
    mesh=_MESH,
    scratch_types=[
        pltpu.VMEM((CH,), jnp.int32),
        pltpu.VMEM((CH,), jnp.int32),
        pltpu.VMEM((CH, 16), jnp.float32),
        pltpu.VMEM((CH, 128), jnp.float32),
        pltpu.VMEM((CH, 128), jnp.float32),
        pltpu.VMEM_SHARED((NP, 128), jnp.float32),
        pltpu.SemaphoreType.DMA,
    ],
)(_sc_gat_a_body)


def _sc_gat_b_body(src, dst, exw, ew, xph, zw, part_out,
                   idx_s, idx_d, ex_v, w_v, rows_v, msg_v, acc, sem):
    c = lax.axis_index("c")
    s = lax.axis_index("s")
    w = c * 16 + s
    _zero_acc(zw, acc, s)
    plsc.subcore_barrier()

    lane = lax.iota(jnp.int32, 16)
    z16 = jnp.zeros((16,), jnp.float32)

    def zrow(e, cc):
        for kk in range(5, 8):
            msg_v[e, pl.ds(kk * 16, 16)] = z16
        return cc

    lax.fori_loop(0, CH, zrow, 0)

    def chunk(g, carry):
        base = w * EPW + g * CH
        pltpu.sync_copy(src.at[pl.ds(base, CH)], idx_s)
        pltpu.sync_copy(dst.at[pl.ds(base, CH)], idx_d)
        pltpu.sync_copy(exw.at[pl.ds(base, CH)], ex_v)
        pltpu.sync_copy(ew.at[pl.ds(base, CH)], w_v)
        pltpu.async_copy(xph.at[idx_s], rows_v, sem).wait()
        for tt in range(CH // 16):
            wv = w_v[pl.ds(tt * 16, 16)]

            def edge(j, cc):
                e = tt * 16 + j
                exv = ex_v[e, pl.ds(0, 16)]
                for h in range(2):
                    f = _take(exv, 4 + h)
                    o = h * 32
                    msg_v[e, pl.ds(o, 16)] = rows_v[e, pl.ds(o, 16)] * f
                    msg_v[e, pl.ds(o + 16, 16)] = rows_v[e, pl.ds(o + 16, 16)] * f
                den = jnp.where(lane == 6, _take(wv, j), exv)
                msg_v[e, pl.ds(64, 16)] = den
                return cc

            lax.fori_loop(0, 16, edge, 0)
        pltpu.sync_copy(msg_v, acc.at[idx_d], add=True)
        return carry

    lax.fori_loop(0, NCH, chunk, 0)
    plsc.subcore_barrier()
    _copy_out(acc, part_out, c, s)


_sc_gat_b = functools.partial(
    pl.kernel,
    out_type=jax.ShapeDtypeStruct((2, NP, 128), jnp.float32),
    mesh=_MESH,
    scratch_types=[
        pltpu.VMEM((CH,), jnp.int32),
        pltpu.VMEM((CH,), jnp.int32),
        pltpu.VMEM((CH, 16), jnp.float32),
        pltpu.VMEM((CH,), jnp.float32),
        pltpu.VMEM((CH, 128), jnp.float32),
        pltpu.VMEM((CH, 128), jnp.float32),
        pltpu.VMEM_SHARED((NP, 128), jnp.float32),
        pltpu.SemaphoreType.DMA,
    ],
)(_sc_gat_b_body)


def _sc_cat_body(src, dst, ew, tab, z128, part_out,
                 idx_s, idx_d, w_v, rows_v, acc, sem):
    c = lax.axis_index("c")
    s = lax.axis_index("s")
    w = c * 16 + s
    _zero_acc(z128, acc, s)
    plsc.subcore_barrier()

    def chunk(g, carry):
        base = w * EPW + g * CH
        pltpu.sync_copy(src.at[pl.ds(base, CH)], idx_s)
        pltpu.sync_copy(dst.at[pl.ds(base, CH)], idx_d)
        pltpu.sync_copy(ew.at[pl.ds(base, CH)], w_v)
        pltpu.async_copy(tab.at[idx_s], rows_v, sem).wait()
        for t in range(CH // 16):
            wv = w_v[pl.ds(t * 16, 16)]

            def edge(j, cc):
                e = t * 16 + j
                f = _take(wv, j)
                for k in range(6):
                    o = k * 16
                    rows_v[e, pl.ds(o, 16)] = rows_v[e, pl.ds(o, 16)] * f
                return cc

            lax.fori_loop(0, 16, edge, 0)
        pltpu.sync_copy(rows_v, acc.at[idx_d], add=True)
        return carry

    lax.fori_loop(0, NCH, chunk, 0)
    plsc.subcore_barrier()
    _copy_out(acc, part_out, c, s)


_sc_cat = functools.partial(
    pl.kernel,
    out_type=jax.ShapeDtypeStruct((2, NP, 128), jnp.float32),
    mesh=_MESH,
    scratch_types=[
        pltpu.VMEM((CH,), jnp.int32),
        pltpu.VMEM((CH,), jnp.int32),
        pltpu.VMEM((CH,), jnp.float32),
        pltpu.VMEM((CH, 128), jnp.float32),
        pltpu.VMEM_SHARED((NP, 128), jnp.float32),
        pltpu.SemaphoreType.DMA,
    ],
)(_sc_cat_body)


_BLK = 1000
_GRID = N // _BLK


def _full(shape):
    return pl.BlockSpec(shape, lambda i: tuple(0 for _ in shape))


def _rows(width):
    return pl.BlockSpec((_BLK, width), lambda i: (i, 0))


def _rows3(width):
    return pl.BlockSpec((2, _BLK, width), lambda i: (0, i, 0))


def _tc1_body(x, wpre, bpre, gatw, am, wcat,
              xpa_o, xpb_o, atab_o, exself_o, xp2_o):
    h = jnp.dot(x[...], wpre[...], preferred_element_type=jnp.float32) + bpre[...]
    xp = jnp.dot(h, gatw[...], preferred_element_type=jnp.float32)
    atab = jnp.dot(xp, am[...], preferred_element_type=jnp.float32)
    al = atab[:, 0:6] + atab[:, 6:12]
    ex6 = jnp.exp(jnp.where(al > 0, al, al * 0.2))
    exself_o[...] = jnp.pad(ex6, ((0, 0), (0, 10)))
    xpa_o[...] = xp[:, 0:128]
    xpb_o[...] = jnp.pad(xp[:, 128:HO], ((0, 0), (0, 64)))
    atab_o[...] = atab
    xp2_o[...] = jnp.dot(h, wcat[...], preferred_element_type=jnp.float32)


def _tc3_body(pga, pgb, exself, xpa, xpb, xp2, gat_b, p16, initw, rootw,
              h1_o, cat_o, root_o, dent_o):
    pgb_s = pgb[0] + pgb[1]
    den16 = jnp.pad(pgb_s[:, 64:80], ((0, 0), (0, 0)))
    deg0 = den16[:, 6:7]
    dis_g = lax.rsqrt(deg0 + 1.0)
    dis_a = jnp.where(deg0 > 0, lax.rsqrt(jnp.where(deg0 > 0, deg0, 1.0)), 0.0)
    ex_exp = jnp.dot(exself[...], p16[...], preferred_element_type=jnp.float32)
    den_exp = jnp.dot(den16, p16[...], preferred_element_type=jnp.float32)
    pg = jnp.concatenate([pga[0] + pga[1], pgb_s[:, 0:64]], axis=1)
    xp = jnp.concatenate([xpa[...], xpb[:, 0:64]], axis=1)
    num = pg + ex_exp * xp
    den = den_exp + ex_exp + 1e-16
    g = num / den + gat_b[...]
    h1 = jnp.where(g > 0, g, g * 0.01)
    h1_o[...] = h1
    xpa = jnp.dot(h1, initw[...], preferred_element_type=jnp.float32) * dis_a
    cat_o[...] = jnp.concatenate(
        [xp2[...] * dis_g, xpa, jnp.zeros((_BLK, 32), jnp.float32)], axis=1)
    root_o[...] = jnp.dot(h1, rootw[...], preferred_element_type=jnp.float32)
    dent_o[...] = jnp.concatenate(
        [dis_g, dis_a, jnp.zeros((_BLK, 6), jnp.float32)], axis=1)


def _tc4_body(h1, pc, dent, xp2, bcat, root, arma_b, out_o):
    dis_g = dent[:, 0:1]
    dis_a = dent[:, 1:2]
    csum = pc[0] + pc[1]
    gs = csum[:, 0:64] * dis_g + xp2[...] * (dis_g * dis_g) + bcat[...]
    h23 = jnp.where(gs > 0, gs, gs * 0.01)
    asum = csum[:, 64:96] * dis_a + root[...] + arma_b[...]
    h4 = jnp.maximum(asum, 0.0)
    out_o[...] = jnp.tanh(jnp.concatenate([h1[...], h23, h4], axis=1))


def kernel(x, edge_index, edge_weight, W_pre, b_pre, gat_W, att_src, att_dst, gat_b,
           gcn0_W, gcn0_b, gcn1_W, gcn1_b, arma_init_W, arma_root_W, arma_b):
    f32 = jnp.float32
    src = edge_index[0].astype(jnp.int32)
    dst = edge_index[1].astype(jnp.int32)

    eye6c = jnp.repeat(jnp.eye(6, dtype=f32), 32, axis=0)
    am = jnp.concatenate(
        [eye6c * att_src.reshape(HO, 1), eye6c * att_dst.reshape(HO, 1),
         jnp.zeros((HO, 116), f32)], axis=1)
    p16 = jnp.pad(jnp.repeat(jnp.eye(6, dtype=f32), 32, axis=1),
                  ((0, 10), (0, 0)))
    wcat = jnp.concatenate([gcn0_W, gcn1_W], axis=1)
    bcat = jnp.concatenate([gcn0_b, gcn1_b]).reshape(1, 64)

    xpa_t, xpb_t, atab, exself, xp2cat = pl.pallas_call(
        _tc1_body,
        grid=(_GRID,),
        in_specs=[_rows(CUR), _full((CUR, HID)), _full((1, HID)),
                  _full((HID, HO)), _full((HO, 128)), _full((HID, 64))],
        out_specs=[_rows(128), _rows(128), _rows(128), _rows(16), _rows(64)],
        out_shape=[jax.ShapeDtypeStruct((N, 128), f32),
                   jax.ShapeDtypeStruct((N, 128), f32),
                   jax.ShapeDtypeStruct((N, 128), f32),
                   jax.ShapeDtypeStruct((N, 16), f32),
                   jax.ShapeDtypeStruct((N, 64), f32)],
    )(x, W_pre, b_pre.reshape(1, HID), gat_W, am, wcat)

    USE_SC = (True, True, True)

    exw = _sc_edge1(src, dst, atab)
    if not USE_SC[0]:
        al_e = atab[src, 0:6] + atab[dst, 6:12]
        al_e = jnp.where(al_e > 0, al_e, al_e * 0.2)
        ex_e = jnp.exp(al_e)
        exw = jnp.concatenate([ex_e, jnp.zeros((E, 10), f32)], axis=1)

    pgat_a = _sc_gat_a(src, dst, exw, xpa_t, jnp.zeros((NP, 128), f32))
    pgat_b = _sc_gat_b(src, dst, exw, edge_weight, xpb_t,
                       jnp.zeros((NP, 128), f32))
    if not USE_SC[1]:
        ex_e = exw[:, 0:6]
        msg = xpa_t[src, 0:128] * jnp.repeat(ex_e[:, 0:4], 32, axis=1)
        pa = jax.ops.segment_sum(msg, dst, num_segments=N)
        den16 = jnp.concatenate([ex_e, edge_weight[:, None],
                                 jnp.zeros((E, 9), f32)], axis=1)
        msgb = jnp.concatenate(
            [xpb_t[src, 0:64] * jnp.repeat(ex_e[:, 4:6], 32, axis=1),
             den16, jnp.zeros((E, 48), f32)], axis=1)
        pb = jax.ops.segment_sum(msgb, dst, num_segments=N)
        pgat_a = jnp.stack([jnp.pad(pa, ((0, NP - N), (0, 0))),
                            jnp.zeros((NP, 128), f32)])
        pgat_b = jnp.stack([jnp.pad(pb, ((0, NP - N), (0, 0))),
                            jnp.zeros((NP, 128), f32)])

    h1, cat_tab, root, dent = pl.pallas_call(
        _tc3_body,
        grid=(_GRID,),
        in_specs=[_rows3(128), _rows3(128), _rows(16), _rows(128),
                  _rows(128), _rows(64),
                  _full((1, HO)), _full((16, HO)),
                  _full((HO, OUT)), _full((HO, OUT))],
        out_specs=[_rows(HO), _rows(128), _rows(OUT), _rows(8)],
        out_shape=[jax.ShapeDtypeStruct((N, HO), f32),
                   jax.ShapeDtypeStruct((N, 128), f32),
                   jax.ShapeDtypeStruct((N, OUT), f32),
                   jax.ShapeDtypeStruct((N, 8), f32)],
    )(pgat_a, pgat_b, exself, xpa_t, xpb_t, xp2cat,
      gat_b.reshape(1, HO), p16, arma_init_W, arma_root_W)

    part_cat = _sc_cat(src, dst, edge_weight, cat_tab, jnp.zeros((NP, 128), f32))
    if not USE_SC[2]:
        pc = jax.ops.segment_sum(cat_tab[src] * edge_weight[:, None],
                                 dst, num_segments=N)
        part_cat = jnp.stack([jnp.pad(pc, ((0, NP - N), (0, 0))),
                              jnp.zeros((NP, 128), f32)])

    out = pl.pallas_call(
        _tc4_body,
        grid=(_GRID,),
        in_specs=[_rows(HO), _rows3(128), _rows(8), _rows(64),
                  _full((1, 64)), _rows(OUT), _full((1, OUT))],
        out_specs=_rows(HO + 3 * OUT),
        out_shape=jax.ShapeDtypeStruct((N, HO + 3 * OUT), f32),
    )(h1, part_cat, dent, xp2cat, bcat, root, arma_b.reshape(1, OUT))
    return out
